# Initial kernel scaffold; baseline (speedup 1.0000x reference)
import jax
import jax.numpy as jnp
from jax.experimental import pallas as pl


def kernel(x, W_qkv, b_qkv, W_out, b_out, routes):
    raise NotImplementedError("write your pallas kernel here")


# speedup vs baseline: 12.1537x; 12.1537x over previous
import functools
import math

import jax
import jax.numpy as jnp
from jax.experimental import pallas as pl
from jax.experimental.pallas import tpu as pltpu

S = 2048
DIM = 1024
NUM_HEADS = 16
HEAD_DIM = DIM // NUM_HEADS
KN = 64
SCALE = 1.0 / math.sqrt(HEAD_DIM)

QB = 256
NB = 512


def _qkv_body(x_ref, w_ref, b_ref, o_ref):
    o_ref[...] = (
        jnp.dot(x_ref[...], w_ref[...], preferred_element_type=jnp.float32)
        + b_ref[...]
    )


def _attn_body(routes_ref, qrow_ref, kv_ref, wout_ref, bout_ref, o_ref):
    r = routes_ref[...]
    cols = jax.lax.broadcasted_iota(jnp.int32, (QB, S), 1)
    mask = jnp.zeros((QB, S), dtype=jnp.bool_)
    for j in range(KN):
        mask = jnp.logical_or(mask, cols == r[:, j][:, None])
    bias = jnp.where(mask, 0.0, -1e30).astype(jnp.float32)

    q_all = qrow_ref[...]
    outs = []
    for h in range(NUM_HEADS):
        q = q_all[:, h * HEAD_DIM:(h + 1) * HEAD_DIM]
        k = kv_ref[:, DIM + h * HEAD_DIM: DIM + (h + 1) * HEAD_DIM]
        v = kv_ref[:, 2 * DIM + h * HEAD_DIM: 2 * DIM + (h + 1) * HEAD_DIM]
        scores = jax.lax.dot_general(
            q, k, (((1,), (1,)), ((), ())), preferred_element_type=jnp.float32
        ) * SCALE + bias
        m = jnp.max(scores, axis=1, keepdims=True)
        e = jnp.exp(scores - m)
        attn = e / jnp.sum(e, axis=1, keepdims=True)
        outs.append(jnp.dot(attn, v, preferred_element_type=jnp.float32))
    attn_out = jnp.concatenate(outs, axis=1)
    o_ref[...] = (
        jnp.dot(attn_out, wout_ref[...], preferred_element_type=jnp.float32)
        + bout_ref[...]
    )


@functools.partial(jax.jit, static_argnames=("interpret",))
def _run(x, W_qkv, b_qkv, W_out, b_out, routes, interpret=False):
    x2 = x.reshape(S, DIM)
    b_qkv2 = b_qkv.reshape(1, 3 * DIM)
    b_out2 = b_out.reshape(1, DIM)

    qkv = pl.pallas_call(
        _qkv_body,
        grid=(S // QB, (3 * DIM) // NB),
        in_specs=[
            pl.BlockSpec((QB, DIM), lambda i, j: (i, 0)),
            pl.BlockSpec((DIM, NB), lambda i, j: (0, j)),
            pl.BlockSpec((1, NB), lambda i, j: (0, j)),
        ],
        out_specs=pl.BlockSpec((QB, NB), lambda i, j: (i, j)),
        out_shape=jax.ShapeDtypeStruct((S, 3 * DIM), jnp.float32),
        interpret=interpret,
    )(x2, W_qkv, b_qkv2)

    out = pl.pallas_call(
        _attn_body,
        grid=(S // QB,),
        in_specs=[
            pl.BlockSpec((QB, KN), lambda i: (i, 0)),
            pl.BlockSpec((QB, 3 * DIM), lambda i: (i, 0)),
            pl.BlockSpec((S, 3 * DIM), lambda i: (0, 0)),
            pl.BlockSpec((DIM, DIM), lambda i: (0, 0)),
            pl.BlockSpec((1, DIM), lambda i: (0, 0)),
        ],
        out_specs=pl.BlockSpec((QB, DIM), lambda i: (i, 0)),
        out_shape=jax.ShapeDtypeStruct((S, DIM), jnp.float32),
        interpret=interpret,
    )(routes, qkv, qkv, W_out, b_out2)

    return out.reshape(1, S, DIM)


def kernel(x, W_qkv, b_qkv, W_out, b_out, routes):
    return _run(x, W_qkv, b_qkv, W_out, b_out, routes)
